# baseline (speedup 1.0000x reference)
import functools

import jax
import jax.numpy as jnp
from jax import lax
from jax.experimental import pallas as pl
from jax.experimental.pallas import tpu as pltpu
from jax.experimental.pallas import tpu_sc as plsc

B = 4096
H = 50
V = 100000
D = 64

NC = 2
NS = 16
NW = NC * NS
RPW = B // NW
NBUF = 5
GLA = 2
L = 16

_mesh = plsc.VectorSubcoreMesh(core_axis_name="c", subcore_axis_name="s")


@functools.partial(
    pl.kernel,
    mesh=_mesh,
    out_type=jax.ShapeDtypeStruct((H, D, B), jnp.float32),
    scratch_types=[
        pltpu.VMEM((RPW, H), jnp.int32),
        pltpu.VMEM((H, RPW), jnp.int32),
        pltpu.VMEM((NBUF, RPW, D), jnp.float32),
        pltpu.VMEM((NBUF, D, RPW), jnp.float32),
        pltpu.SemaphoreType.DMA,
        pltpu.SemaphoreType.DMA,
    ],
    compiler_params=pltpu.CompilerParams(
        use_tc_tiling_on_sc=False,
        needs_layout_passes=False,
        disable_bounds_checks=True,
    ),
)
def _emb_lookup(idx_hbm, w_hbm, out_hbm, idx_v, idxT_v, rows_v, blk_v, gsem, ssem):
    wid = lax.axis_index("s") * NC + lax.axis_index("c")
    row0 = wid * RPW

    pltpu.sync_copy(idx_hbm.at[pl.ds(row0, RPW)], idx_v)

    rvecs = [lax.iota(jnp.int32, L) + L * k for k in range(RPW // L)]

    def tidx_body(h, carry):
        cvec = jnp.full((L,), h, dtype=jnp.int32)
        for k in range(RPW // L):
            v = plsc.load_gather(idx_v, [rvecs[k], cvec])
            idxT_v[h, pl.ds(L * k, L)] = v
        return carry

    lax.fori_loop(0, H, tidx_body, 0)

    def issue_gather(h, slot):
        pltpu.async_copy(w_hbm.at[idxT_v.at[h]], rows_v.at[slot], gsem)

    def wait_gather(h, slot):
        pltpu.make_async_copy(
            w_hbm.at[idxT_v.at[h]], rows_v.at[slot], gsem
        ).wait()

    def issue_store(h, slot):
        pltpu.async_copy(
            blk_v.at[slot], out_hbm.at[h, :, pl.ds(row0, RPW)], ssem
        )

    def wait_one_store():
        pltpu.make_async_copy(
            blk_v.at[0], out_hbm.at[0, :, pl.ds(row0, RPW)], ssem
        ).wait()

    XP_UNROLL = 4

    def transpose_block(slot):
        def xp_body(i, carry):
            d0 = i * XP_UNROLL
            for dd in range(XP_UNROLL):
                cvec = jnp.full((L,), d0 + dd, dtype=jnp.int32)
                for k in range(RPW // L):
                    v = plsc.load_gather(rows_v.at[slot], [rvecs[k], cvec])
                    blk_v[slot, d0 + dd, pl.ds(L * k, L)] = v
            return carry

        lax.fori_loop(0, D // XP_UNROLL, xp_body, 0)

    def step(h, s, store_wait, issue):
        if store_wait:
            wait_one_store()
        if issue:
            issue_gather(h + GLA, (h + GLA) % NBUF)
        wait_gather(h, s)
        transpose_block(s)
        issue_store(h, s)

    for h in range(GLA):
        issue_gather(h, h)

    for s in range(NBUF):
        step(s, s, store_wait=False, issue=True)

    def outer(t, carry):
        for s in range(NBUF):
            step(t * NBUF + s, s, store_wait=True, issue=True)
        return carry

    lax.fori_loop(1, H // NBUF - 1, outer, 0)

    for s in range(NBUF):
        h = H - NBUF + s
        step(h, s, store_wait=True, issue=(h + GLA < H))

    for _ in range(NBUF):
        wait_one_store()


def kernel(input_ids, weight):
    out_hdb = _emb_lookup(input_ids.astype(jnp.int32), weight)
    return jnp.transpose(out_hdb, (2, 0, 1))


# speedup vs baseline: 1.8291x; 1.8291x over previous
import functools

import jax
import jax.numpy as jnp
from jax import lax
from jax.experimental import pallas as pl
from jax.experimental.pallas import tpu as pltpu
from jax.experimental.pallas import tpu_sc as plsc

B = 4096
H = 50
V = 100000
D = 64

NC = 2
NS = 16
NW = NC * NS
RPW = B // NW
CR = 1
NCHK = RPW // CR
NBUF = 16
LA = 8
T = NCHK // NBUF

_mesh = plsc.VectorSubcoreMesh(core_axis_name="c", subcore_axis_name="s")


@functools.partial(
    pl.kernel,
    mesh=_mesh,
    out_type=jax.ShapeDtypeStruct((B, H, D), jnp.float32),
    scratch_types=[
        pltpu.VMEM((RPW, H), jnp.int32),
        pltpu.VMEM((NBUF, H, D), jnp.float32),
        pltpu.SemaphoreType.DMA,
        pltpu.SemaphoreType.DMA,
    ],
    compiler_params=pltpu.CompilerParams(
        use_tc_tiling_on_sc=False,
        needs_layout_passes=False,
        disable_bounds_checks=True,
    ),
)
def _emb_lookup(idx_hbm, w_hbm, out_hbm, idx_v, rows_v, gsem, ssem):
    wid = lax.axis_index("s") * NC + lax.axis_index("c")
    row0 = wid * RPW

    pltpu.sync_copy(idx_hbm.at[pl.ds(row0, RPW)], idx_v)

    def issue_gather(c, slot):
        pltpu.async_copy(w_hbm.at[idx_v.at[c]], rows_v.at[slot], gsem)

    def wait_gather(c, slot):
        pltpu.make_async_copy(
            w_hbm.at[idx_v.at[c]], rows_v.at[slot], gsem
        ).wait()

    def issue_store(c, slot):
        pltpu.async_copy(rows_v.at[slot], out_hbm.at[row0 + c], ssem)

    def wait_one_store():
        pltpu.make_async_copy(rows_v.at[0], out_hbm.at[row0], ssem).wait()

    def step(c, b, store_wait, issue):
        if store_wait:
            wait_one_store()
        if issue:
            issue_gather(c + LA, (b + LA) % NBUF)
        wait_gather(c, b)
        issue_store(c, b)

    for c in range(LA):
        issue_gather(c, c)

    for b in range(NBUF):
        step(b, b, store_wait=(b >= NBUF - LA), issue=True)

    def outer(t, carry):
        for b in range(NBUF):
            step(t * NBUF + b, b, store_wait=True, issue=True)
        return carry

    lax.fori_loop(1, T - 1, outer, 0)

    for b in range(NBUF):
        c = (T - 1) * NBUF + b
        step(c, b, store_wait=True, issue=(c + LA < NCHK))

    for _ in range(NBUF - LA):
        wait_one_store()


def kernel(input_ids, weight):
    return _emb_lookup(input_ids.astype(jnp.int32), weight)
